# baseline (speedup 1.0000x reference)
import jax
import jax.numpy as jnp
from jax.experimental import pallas as pl
from jax.experimental.pallas import tpu as pltpu

HIDDEN = 4096
NUM_EXPERTS = 64
TOP_K = 8
BT = 1024
BK = 1024
NK = HIDDEN // BK


def _router_block(x_ref, w_ref, idx_ref, gate_ref, acc_ref):
    j = pl.program_id(1)
    part = jax.lax.dot_general(
        x_ref[...], w_ref[...],
        dimension_numbers=(((1,), (1,)), ((), ())),
        preferred_element_type=jnp.float32,
    )

    @pl.when(j == 0)
    def _init():
        acc_ref[...] = part

    @pl.when(j > 0)
    def _acc():
        acc_ref[...] += part

    @pl.when(j == NK - 1)
    def _epilogue():
        lt = acc_ref[...].T
        iota = jax.lax.broadcasted_iota(jnp.int32, lt.shape, 0)
        vals = lt
        top_vals, top_idx = [], []
        for _ in range(TOP_K):
            m = jnp.max(vals, axis=0, keepdims=True)
            idx = jnp.min(jnp.where(vals == m, iota, NUM_EXPERTS), axis=0,
                          keepdims=True)
            top_vals.append(m)
            top_idx.append(idx)
            vals = jnp.where(iota == idx, -jnp.inf, vals)
        tv = jnp.concatenate(top_vals, axis=0)
        ti = jnp.concatenate(top_idx, axis=0)
        e = jnp.exp(tv - tv[0:1, :])
        gate_ref[...] = e / jnp.sum(e, axis=0, keepdims=True)
        idx_ref[...] = ti


def kernel(x, W):
    tokens = x.shape[0]
    grid = (tokens // BT, NK)
    idx_t, gates_t = pl.pallas_call(
        _router_block,
        grid=grid,
        in_specs=[
            pl.BlockSpec((BT, BK), lambda i, j: (i, j)),
            pl.BlockSpec((NUM_EXPERTS, BK), lambda i, j: (0, j)),
        ],
        out_specs=[
            pl.BlockSpec((TOP_K, BT), lambda i, j: (0, i)),
            pl.BlockSpec((TOP_K, BT), lambda i, j: (0, i)),
        ],
        out_shape=[
            jax.ShapeDtypeStruct((TOP_K, tokens), jnp.int32),
            jax.ShapeDtypeStruct((TOP_K, tokens), jnp.float32),
        ],
        scratch_shapes=[pltpu.VMEM((BT, NUM_EXPERTS), jnp.float32)],
    )(x, W)
    return idx_t.T, gates_t.T


# speedup vs baseline: 1.4038x; 1.4038x over previous
import jax
import jax.numpy as jnp
from jax.experimental import pallas as pl

HIDDEN = 4096
NUM_EXPERTS = 64
TOP_K = 8
BT = 1024


def _router_block(x_ref, w_ref, out_ref):
    logits = jax.lax.dot_general(
        x_ref[...], w_ref[...],
        dimension_numbers=(((1,), (1,)), ((), ())),
        preferred_element_type=jnp.float32,
    )
    lt = logits.T
    iota = jax.lax.broadcasted_iota(jnp.int32, lt.shape, 0)
    vals = lt
    top_vals, top_idx = [], []
    for _ in range(TOP_K):
        m = jnp.max(vals, axis=0, keepdims=True)
        idx = jnp.min(jnp.where(vals == m, iota, NUM_EXPERTS), axis=0,
                      keepdims=True)
        top_vals.append(m)
        top_idx.append(idx)
        vals = jnp.where(iota == idx, -jnp.inf, vals)
    tv = jnp.concatenate(top_vals, axis=0)
    e = jnp.exp(tv - tv[0:1, :])
    g = e / jnp.sum(e, axis=0, keepdims=True)
    ti = [t.astype(jnp.float32) for t in top_idx]
    out_ref[...] = jnp.concatenate(ti + [g], axis=0)


def kernel(x, W):
    tokens = x.shape[0]
    grid = (tokens // BT,)
    packed = pl.pallas_call(
        _router_block,
        grid=grid,
        in_specs=[
            pl.BlockSpec((BT, HIDDEN), lambda i: (i, 0)),
            pl.BlockSpec((NUM_EXPERTS, HIDDEN), lambda i: (0, 0)),
        ],
        out_specs=pl.BlockSpec((2 * TOP_K, BT), lambda i: (0, i)),
        out_shape=jax.ShapeDtypeStruct((2 * TOP_K, tokens), jnp.float32),
    )(x, W)
    p = packed.T
    return p[:, :TOP_K].astype(jnp.int32), p[:, TOP_K:]


# speedup vs baseline: 1.4288x; 1.0178x over previous
import jax
import jax.numpy as jnp
from jax.experimental import pallas as pl

HIDDEN = 4096
NUM_EXPERTS = 64
TOP_K = 8
BT = 1024


def _router_block(x1_ref, x2_ref, w_ref, idx_ref, gate_ref):
    dn = (((1,), (1,)), ((), ()))
    logits = jax.lax.dot_general(
        x1_ref[...], w_ref[:, : HIDDEN // 2],
        dimension_numbers=dn, preferred_element_type=jnp.float32,
    ) + jax.lax.dot_general(
        x2_ref[...], w_ref[:, HIDDEN // 2 :],
        dimension_numbers=dn, preferred_element_type=jnp.float32,
    )
    lt = logits.T
    iota = jax.lax.broadcasted_iota(jnp.int32, lt.shape, 0)
    vals = lt
    top_vals, top_idx = [], []
    for _ in range(TOP_K):
        m = jnp.max(vals, axis=0, keepdims=True)
        idx = jnp.min(jnp.where(vals == m, iota, NUM_EXPERTS), axis=0,
                      keepdims=True)
        top_vals.append(m)
        top_idx.append(idx)
        vals = jnp.where(iota == idx, -jnp.inf, vals)
    tv = jnp.concatenate(top_vals, axis=0)
    ti = jnp.concatenate(top_idx, axis=0)
    e = jnp.exp(tv - tv[0:1, :])
    gate_ref[...] = e / jnp.sum(e, axis=0, keepdims=True)
    idx_ref[...] = ti


def kernel(x, W):
    tokens = x.shape[0]
    grid = (tokens // BT,)
    idx_t, gates_t = pl.pallas_call(
        _router_block,
        grid=grid,
        in_specs=[
            pl.BlockSpec((BT, HIDDEN // 2), lambda i: (i, 0)),
            pl.BlockSpec((BT, HIDDEN // 2), lambda i: (i, 1)),
            pl.BlockSpec((NUM_EXPERTS, HIDDEN), lambda i: (0, 0)),
        ],
        out_specs=[
            pl.BlockSpec((TOP_K, BT), lambda i: (0, i)),
            pl.BlockSpec((TOP_K, BT), lambda i: (0, i)),
        ],
        out_shape=[
            jax.ShapeDtypeStruct((TOP_K, tokens), jnp.int32),
            jax.ShapeDtypeStruct((TOP_K, tokens), jnp.float32),
        ],
    )(x, x, W)
    return idx_t.T, gates_t.T
